# baseline (speedup 1.0000x reference)
import functools

import jax
import jax.numpy as jnp
from jax import lax
from jax.experimental import pallas as pl
from jax.experimental.pallas import tpu as pltpu
from jax.experimental.pallas import tpu_sc as plsc

EMBED = 64
BATCH = 16384
OUT_W = 5 * EMBED

_info = plsc.get_sparse_core_info()
_NW = _info.num_cores * _info.num_subcores
_BPW = BATCH // _NW
_CH = 128
_NCH = _BPW // _CH

_mesh = plsc.VectorSubcoreMesh(core_axis_name="c", subcore_axis_name="s")
_params = pltpu.CompilerParams(use_tc_tiling_on_sc=True,
                               needs_layout_passes=False)


def _lookup_body(idx_hbm, tables, out, idx_v, rows_v, stage_v, sem, row0):
    nf = len(tables)
    wid = lax.axis_index("s") * _info.num_cores + lax.axis_index("c")
    base = wid * _BPW
    for t in range(nf):
        pltpu.sync_copy(idx_hbm[t].at[pl.ds(base, _BPW)], idx_v[t])

    lanes = jax.lax.iota(jnp.int32, 16)

    def issue(g, _, t, c, buf):
        v = idx_v[t][pl.ds(c * _CH + g * 16, 16)]
        for lane in range(16):
            pltpu.async_copy(
                tables[t].at[pl.ds(v[lane], 1), :],
                rows_v[buf].at[pl.ds(g * 16 + lane, 1), :],
                sem[buf])
        return 0

    def drain(g, _, buf):
        for lane in range(16):
            pltpu.make_async_copy(
                tables[0].at[pl.ds(0, 1), :],
                rows_v[0].at[pl.ds(0, 1), :],
                sem[buf]).wait()
        return 0

    def chunk(c, _):
        def field(t, buf):
            lax.fori_loop(0, _CH // 16,
                          functools.partial(issue, t=t, c=c, buf=buf), 0)

        field(0, 0)

        for t in range(nf):
            lax.fori_loop(0, _CH // 16,
                          functools.partial(drain, buf=t % 2), 0)
            if t + 1 < nf:
                field(t + 1, (t + 1) % 2)

            def asm(u, _, t=t, buf=t % 2):
                for d0 in range(0, EMBED, 16):
                    v = rows_v[buf][u, pl.ds(d0, 16)]
                    plsc.store_scatter(
                        stage_v,
                        [t * EMBED + d0 + lanes, jnp.full((16,), u, jnp.int32)],
                        v)
                return 0
            lax.fori_loop(0, _CH, asm, 0)

        pltpu.sync_copy(
            stage_v,
            out.at[pl.ds(row0, nf * EMBED), pl.ds(base + c * _CH, _CH)])
        return 0

    lax.fori_loop(0, _NCH, chunk, 0)


@functools.partial(
    pl.kernel,
    mesh=_mesh,
    out_type=(),
    scratch_types=[
        [pltpu.VMEM((_BPW,), jnp.int32) for _ in range(4)],
        [pltpu.VMEM((_CH, EMBED), jnp.float32) for _ in range(2)],
        pltpu.VMEM((4 * EMBED, _CH), jnp.float32),
        [pltpu.SemaphoreType.DMA for _ in range(2)],
    ],
    compiler_params=_params,
    name="small_fields",
)
def _small_fields(ep, pop, yr, st, et, pt, yt, stt, out,
                  idx_v, rows_v, stage_v, sem):
    _lookup_body([ep, pop, yr, st], [et, pt, yt, stt], out,
                 idx_v, rows_v, stage_v, sem, EMBED)


@functools.partial(
    pl.kernel,
    mesh=_mesh,
    out_type=(),
    scratch_types=[
        [pltpu.VMEM((_BPW,), jnp.int32)],
        [pltpu.VMEM((_CH, EMBED), jnp.float32) for _ in range(2)],
        pltpu.VMEM((EMBED, _CH), jnp.float32),
        [pltpu.SemaphoreType.DMA for _ in range(2)],
    ],
    compiler_params=_params,
    name="user_field",
)
def _user_field(uid, ut, out, idx_v, rows_v, stage_v, sem):
    _lookup_body([uid], [ut], out, idx_v, rows_v, stage_v, sem, 0)


def kernel(user_id, episodes, popularity, year, studio,
           user_table, episodes_table, popularity_table, year_table, studio_table):
    o_ref = jax.new_ref(pl.empty((OUT_W, BATCH), jnp.float32))
    _small_fields(episodes, popularity, year, studio,
                  episodes_table, popularity_table, year_table, studio_table,
                  o_ref)
    _user_field(user_id, user_table, o_ref)
    return o_ref[...].T
